# baseline (speedup 1.0000x reference)
import functools

import jax
import jax.numpy as jnp
from jax import lax
from jax.experimental import pallas as pl
from jax.experimental.pallas import tpu as pltpu
from jax.experimental.pallas import tpu_sc as plsc

_KEEP_RATIO = 0.5
_ROWS = 1024
_CHUNK = 256
_SC_ROWS = 3072
_SC_CH = 16
_SC_UNROLL = 8
_NW = 32


def _sc_sumsq_body(rows_w, x_hbm, o_hbm, xa0, xa1, acc, sem0, sem1):
    D = acc.shape[0]
    wid = lax.axis_index("s") * 2 + lax.axis_index("c")
    base = wid * rows_w
    nch = rows_w // _SC_CH
    bufs = (xa0, xa1)
    sems = (sem0, sem1)

    def zero_body(j, _):
        acc[pl.ds(j * 16, 16)] = jnp.zeros((16,), jnp.float32)
        return 0
    lax.fori_loop(0, D // 16, zero_body, 0)

    def copy(ci, buf, sem):
        return pltpu.make_async_copy(
            x_hbm.at[pl.ds(base + ci * _SC_CH, _SC_CH), :], buf, sem)

    copy(0, bufs[0], sems[0]).start()
    for ci in range(nch):
        buf = bufs[ci % 2]
        copy(ci, buf, sems[ci % 2]).wait()
        if ci + 1 < nch:
            copy(ci + 1, bufs[(ci + 1) % 2], sems[(ci + 1) % 2]).start()

        def j_body(j, _, buf=buf):
            jo = j * 16
            a = jnp.zeros((16,), jnp.float32)

            def r_body(rb, a, buf=buf, jo=jo):
                for u in range(_SC_UNROLL):
                    v = buf[rb * _SC_UNROLL + u, pl.ds(jo, 16)]
                    a = a + v * v
                return a
            a = lax.fori_loop(0, _SC_CH // _SC_UNROLL, r_body, a)
            plsc.addupdate(acc.at[pl.ds(jo, 16)], a)
            return 0
        lax.fori_loop(0, D // 16, j_body, 0)
    pltpu.sync_copy(acc, o_hbm.at[wid])


def _tc1_body(keep, row_ref, col_ref, x_ref, coef_ref,
              acc_ref, mask_ref, ab_ref):
    i = pl.program_id(0)
    D = row_ref.shape[1]
    deg = coef_ref.shape[1]

    @pl.when(i == 0)
    def _mask_and_init():
        row = row_ref[...]
        kidx0 = jax.lax.broadcasted_iota(jnp.int32, (_CHUNK, D), 0)
        qidx = jax.lax.broadcasted_iota(jnp.int32, (_CHUNK, D), 1)
        rank = jnp.zeros((1, D), jnp.float32)
        for c in range(D // _CHUNK):
            col = col_ref[pl.ds(c * _CHUNK, _CHUNK), :]
            kidx = kidx0 + c * _CHUNK
            beat = jnp.logical_or(
                col > row,
                jnp.logical_and(col == row, kidx < qidx))
            rank = rank + jnp.sum(jnp.where(beat, 1.0, 0.0),
                                  axis=0, keepdims=True)
        m = rank < keep
        mask_ref[...] = jnp.where(m, 1.0, 0.0)
        for k in range(deg):
            ab_ref[k:k + 1, :] = jnp.where(
                m, coef_ref[0, k], 1.0 if k == 0 else 0.0)
        acc_ref[...] = jnp.zeros((1, D), jnp.float32)

    xb = x_ref[...]
    acc_ref[...] = acc_ref[...] + jnp.sum(xb * xb, axis=0, keepdims=True)


def _tc2_body(x_ref, scpart_ref, acc1_ref, mask_ref, ab_ref, o_ref, loss_ref):
    i = pl.program_id(0)
    deg = ab_ref.shape[0]

    @pl.when(i == 0)
    def _loss():
        tot = acc1_ref[...] + jnp.sum(scpart_ref[...], axis=0, keepdims=True)
        loss_ref[0, 0] = 1e-6 * jnp.sqrt(
            jnp.sum(tot * (1.0 - mask_ref[...])))

    x = x_ref[...]
    p = ab_ref[deg - 1:deg, :] * x
    for k in range(deg - 2, -1, -1):
        p = (p + ab_ref[k:k + 1, :]) * x
    o_ref[...] = p + loss_ref[0, 0]


def kernel(x, coeffs, importance):
    B, T, D = x.shape
    keep = max(1, int(D * _KEEP_RATIO))
    deg = coeffs.shape[0]
    n = B * T
    xf = x.reshape(n, D)
    nsteps1 = (n - _SC_ROWS) // _ROWS
    nsteps2 = n // _ROWS

    scpart = pl.kernel(
        functools.partial(_sc_sumsq_body, _SC_ROWS // _NW),
        out_type=jax.ShapeDtypeStruct((_NW, D), jnp.float32),
        mesh=plsc.VectorSubcoreMesh(core_axis_name="c", subcore_axis_name="s"),
        scratch_types=[
            pltpu.VMEM((_SC_CH, D), jnp.float32),
            pltpu.VMEM((_SC_CH, D), jnp.float32),
            pltpu.VMEM((D,), jnp.float32),
            pltpu.SemaphoreType.DMA,
            pltpu.SemaphoreType.DMA,
        ],
    )(xf)

    acc1, mask, ab = pl.pallas_call(
        functools.partial(_tc1_body, keep),
        grid=(nsteps1,),
        in_specs=[
            pl.BlockSpec((1, D), lambda i: (0, 0)),
            pl.BlockSpec((D, 1), lambda i: (0, 0)),
            pl.BlockSpec((_ROWS, D), lambda i: (_SC_ROWS // _ROWS + i, 0)),
            pl.BlockSpec(memory_space=pltpu.SMEM),
        ],
        out_specs=[
            pl.BlockSpec((1, D), lambda i: (0, 0)),
            pl.BlockSpec((1, D), lambda i: (0, 0)),
            pl.BlockSpec((deg, D), lambda i: (0, 0)),
        ],
        out_shape=[
            jax.ShapeDtypeStruct((1, D), jnp.float32),
            jax.ShapeDtypeStruct((1, D), jnp.float32),
            jax.ShapeDtypeStruct((deg, D), jnp.float32),
        ],
    )(importance.reshape(1, D), importance.reshape(D, 1), xf,
      coeffs.reshape(1, deg))

    y = pl.pallas_call(
        _tc2_body,
        grid=(nsteps2,),
        in_specs=[
            pl.BlockSpec((_ROWS, D), lambda i: (i, 0)),
            pl.BlockSpec((_NW, D), lambda i: (0, 0)),
            pl.BlockSpec((1, D), lambda i: (0, 0)),
            pl.BlockSpec((1, D), lambda i: (0, 0)),
            pl.BlockSpec((deg, D), lambda i: (0, 0)),
        ],
        out_specs=pl.BlockSpec((_ROWS, D), lambda i: (i, 0)),
        out_shape=jax.ShapeDtypeStruct((n, D), jnp.float32),
        scratch_shapes=[pltpu.SMEM((1, 1), jnp.float32)],
    )(xf, scpart, acc1, mask, ab)

    return y.reshape(B, T, D)


# speedup vs baseline: 1.1236x; 1.1236x over previous
import functools

import jax
import jax.numpy as jnp
from jax.experimental import pallas as pl
from jax.experimental.pallas import tpu as pltpu

_KEEP_RATIO = 0.5
_ROWS = 1024
_CHUNK = 256


def _fused_kernel(keep, nsteps, row_ref, col_ref, x_ref, coef_ref,
                  o_ref, acc_ref, mask_ref, ab_ref, loss_ref):
    i = pl.program_id(0)
    D = row_ref.shape[1]
    deg = coef_ref.shape[1]

    @pl.when(i == 0)
    def _mask_and_init():
        row = row_ref[...]
        kidx0 = jax.lax.broadcasted_iota(jnp.int32, (_CHUNK, D), 0)
        qidx = jax.lax.broadcasted_iota(jnp.int32, (_CHUNK, D), 1)
        rank = jnp.zeros((1, D), jnp.float32)
        for c in range(D // _CHUNK):
            col = col_ref[pl.ds(c * _CHUNK, _CHUNK), :]
            kidx = kidx0 + c * _CHUNK
            beat = jnp.logical_or(
                col > row,
                jnp.logical_and(col == row, kidx < qidx))
            rank = rank + jnp.sum(jnp.where(beat, 1.0, 0.0),
                                  axis=0, keepdims=True)
        m = rank < keep
        mask_ref[...] = jnp.where(m, 1.0, 0.0)
        for k in range(deg):
            ab_ref[k:k + 1, :] = jnp.where(
                m, coef_ref[0, k], 1.0 if k == 0 else 0.0)
        acc_ref[...] = jnp.zeros((1, D), jnp.float32)

    @pl.when(i < nsteps)
    def _phase1():
        xb = x_ref[...]
        acc_ref[...] = acc_ref[...] + jnp.sum(xb * xb, axis=0, keepdims=True)

    @pl.when(i == nsteps - 1)
    def _loss():
        loss_ref[0, 0] = 1e-6 * jnp.sqrt(
            jnp.sum(acc_ref[...] * (1.0 - mask_ref[...])))

    @pl.when(i >= nsteps)
    def _phase2():
        x = x_ref[...]
        p = ab_ref[deg - 1:deg, :] * x
        for k in range(deg - 2, -1, -1):
            p = (p + ab_ref[k:k + 1, :]) * x
        o_ref[...] = p + loss_ref[0, 0]


def kernel(x, coeffs, importance):
    B, T, D = x.shape
    keep = max(1, int(D * _KEEP_RATIO))
    deg = coeffs.shape[0]
    n = B * T
    nsteps = n // _ROWS
    xf = x.reshape(n, D)

    y = pl.pallas_call(
        functools.partial(_fused_kernel, keep, nsteps),
        grid=(2 * nsteps,),
        in_specs=[
            pl.BlockSpec((1, D), lambda i: (0, 0)),
            pl.BlockSpec((D, 1), lambda i: (0, 0)),
            pl.BlockSpec((_ROWS, D), lambda i: (i % nsteps, 0)),
            pl.BlockSpec(memory_space=pltpu.SMEM),
        ],
        out_specs=pl.BlockSpec((_ROWS, D), lambda i: (jnp.maximum(i - nsteps, 0), 0)),
        out_shape=jax.ShapeDtypeStruct((n, D), jnp.float32),
        scratch_shapes=[
            pltpu.VMEM((1, D), jnp.float32),
            pltpu.VMEM((1, D), jnp.float32),
            pltpu.VMEM((deg, D), jnp.float32),
            pltpu.SMEM((1, 1), jnp.float32),
        ],
    )(importance.reshape(1, D), importance.reshape(D, 1), xf,
      coeffs.reshape(1, deg))

    return y.reshape(B, T, D)
